# Initial kernel scaffold; baseline (speedup 1.0000x reference)
import jax
import jax.numpy as jnp
from jax.experimental import pallas as pl


def kernel(x, W_enc, W_dec, pre_bias, latent_bias):
    raise NotImplementedError("write your pallas kernel here")


# speedup vs baseline: 1.0001x; 1.0001x over previous
import jax
import jax.numpy as jnp
from jax.experimental import pallas as pl
from jax.experimental.pallas import tpu as pltpu

N, D, H, K = 8192, 1024, 16384, 64
NB = 2048
HB = 1024


def _mm_kernel(x_ref, w_ref, b_ref, o_ref):
    o_ref[...] = jax.lax.dot_general(
        x_ref[...], w_ref[...], (((1,), (1,)), ((), ())),
        preferred_element_type=jnp.float32) + b_ref[...]


def _encode(xc, W_enc, latent_bias):
    return pl.pallas_call(
        _mm_kernel,
        grid=(N // NB, H // HB),
        in_specs=[
            pl.BlockSpec((NB, D), lambda i, j: (i, 0)),
            pl.BlockSpec((HB, D), lambda i, j: (j, 0)),
            pl.BlockSpec((1, HB), lambda i, j: (0, j)),
        ],
        out_specs=pl.BlockSpec((NB, HB), lambda i, j: (i, j)),
        out_shape=jax.ShapeDtypeStruct((N, H), jnp.float32),
    )(xc, W_enc, latent_bias.reshape(1, H))


def kernel(x, W_enc, W_dec, pre_bias, latent_bias):
    xc = x - pre_bias
    pre_acts = _encode(xc, W_enc, latent_bias)
    relu = jnp.maximum(pre_acts, 0.0)
    tv, ti = jax.lax.top_k(relu, K)
    rows = jnp.arange(N)[:, None]
    sparse_code = jnp.zeros_like(pre_acts).at[rows, ti].set(tv)
    recon = sparse_code @ W_dec.T + pre_bias
    return (recon, sparse_code, pre_acts, tv, ti)
